# Initial kernel scaffold; baseline (speedup 1.0000x reference)
import jax
import jax.numpy as jnp
from jax.experimental import pallas as pl


def kernel(x, edge_index, edge_attr, atom_emb, bond_emb, eps, W1, b1, W2, b2, bn_gamma, bn_beta, lin_W, lin_b):
    raise NotImplementedError("write your pallas kernel here")


# speedup vs baseline: 8.2765x; 8.2765x over previous
import functools

import jax
import jax.numpy as jnp
from jax import lax
from jax.experimental import pallas as pl
from jax.experimental.pallas import tpu as pltpu
from jax.experimental.pallas import tpu_sc as plsc

_N = 10000
_E = 320000
_H = 128
_L = 3
_NPAD = 10240
_NC = 2
_NS = 16
_NW = _NC * _NS
_EC = 128
_EP = 327680
_ECH = _EP // (_NW * _EC)
_EGC = 8
_AC = 128
_ACH = -(-(_NPAD // _AC) // _NW)
_TBR = _NPAD // _NS


def _mesh():
    return plsc.VectorSubcoreMesh(
        core_axis_name="c", subcore_axis_name="s",
        num_cores=_NC, num_subcores=_NS)


def _atom_body(xf_hbm, tab_hbm, h0_hbm, tab_sp, idxv, gbuf, hbuf, sem):
    c = lax.axis_index("c")
    s = lax.axis_index("s")
    wid = s * _NC + c

    @pl.when(s == 0)
    def _():
        pltpu.sync_copy(tab_hbm, tab_sp)
    plsc.subcore_barrier()

    for i in range(_ACH):
        ch = wid + _NW * i

        @pl.when(ch < _NPAD // _AC)
        def _():
            pltpu.sync_copy(xf_hbm.at[ch], idxv)

            def zr(k, carry):
                for v in range(_H // 16):
                    hbuf[k, pl.ds(v * 16, 16)] = jnp.zeros((16,), jnp.float32)
                return carry
            lax.fori_loop(0, _AC, zr, 0)
            for j in range(9):
                pltpu.async_copy(tab_sp.at[idxv.at[j]], gbuf, sem).wait()

                def ar(k, carry):
                    for v in range(_H // 16):
                        sl = pl.ds(v * 16, 16)
                        hbuf[k, sl] = hbuf[k, sl] + gbuf[k, sl]
                    return carry
                lax.fori_loop(0, _AC, ar, 0)
            pltpu.sync_copy(hbuf, h0_hbm.at[pl.ds(ch * _AC, _AC)])


def _edge_body(h_hbm, b18_hbm, eidx_hbm, out_hbm,
               tab_sp, acc_sp, idxg, hbuf, ebuf, tabv,
               sem, sem2):
    c = lax.axis_index("c")
    s = lax.axis_index("s")
    wid = s * _NC + c

    def zrow(r, carry):
        for v in range(_H // 16):
            hbuf[r, pl.ds(v * 16, 16)] = jnp.zeros((16,), jnp.float32)
        return carry
    lax.fori_loop(0, _EC, zrow, 0)

    def zcp(i, carry):
        pltpu.sync_copy(hbuf, acc_sp.at[pl.ds(s * _TBR + i * _EC, _EC)])
        return carry
    lax.fori_loop(0, _TBR // _EC, zcp, 0)

    @pl.when(s == 0)
    def _():
        pltpu.sync_copy(b18_hbm, tabv)

        def trow(r, off):
            g = r + off
            a0 = g // 36
            a1 = (g // 6) % 6
            a2 = g % 6
            for v in range(_H // 16):
                sl = pl.ds(v * 16, 16)
                hbuf[r, sl] = (tabv[a0, sl] + tabv[6 + a1, sl]
                               + tabv[12 + a2, sl])
            return off
        lax.fori_loop(0, 112, trow, 0)
        pltpu.sync_copy(hbuf.at[pl.ds(0, 112)], tab_sp.at[pl.ds(0, 112)])
        lax.fori_loop(0, 104, trow, 112)
        pltpu.sync_copy(hbuf.at[pl.ds(0, 104)], tab_sp.at[pl.ds(112, 104)])

    plsc.subcore_barrier()

    def group_body(g, carry):
        pltpu.sync_copy(eidx_hbm.at[wid, g], idxg)
        for il in range(_EGC):
            cp1 = pltpu.async_copy(h_hbm.at[idxg.at[il]], hbuf, sem)
            cp2 = pltpu.async_copy(tab_sp.at[idxg.at[_EGC + il]], ebuf, sem2)
            cp1.wait()
            cp2.wait()

            def erow(k, carry2):
                for v in range(_H // 16):
                    sl = pl.ds(v * 16, 16)
                    hbuf[k, sl] = jnp.maximum(hbuf[k, sl] + ebuf[k, sl], 0.0)
                return carry2
            lax.fori_loop(0, _EC, erow, 0)
            pltpu.sync_copy(hbuf, acc_sp.at[idxg.at[2 * _EGC + il]],
                            add=True)
        return carry
    lax.fori_loop(0, _ECH // _EGC, group_body, 0)

    plsc.subcore_barrier()
    pltpu.sync_copy(acc_sp.at[pl.ds(s * _TBR, _TBR)],
                    out_hbm.at[pl.ds(c * _NPAD + s * _TBR, _TBR)])


def _atom_call(xf2, atab):
    f = pl.kernel(
        _atom_body,
        out_type=jax.ShapeDtypeStruct((_NPAD, _H), jnp.float32),
        mesh=_mesh(),
        scratch_types=[
            pltpu.VMEM_SHARED((900, _H), jnp.float32),
            pltpu.VMEM((16, _AC), jnp.int32),
            pltpu.VMEM((_AC, _H), jnp.float32),
            pltpu.VMEM((_AC, _H), jnp.float32),
            pltpu.SemaphoreType.DMA,
        ])
    return f(xf2, atab)


def _edge_call(h, b18, eidx):
    f = pl.kernel(
        _edge_body,
        out_type=jax.ShapeDtypeStruct((_NC * _NPAD, _H), jnp.float32),
        mesh=_mesh(),
        scratch_types=[
            pltpu.VMEM_SHARED((216, _H), jnp.float32),
            pltpu.VMEM_SHARED((_NPAD, _H), jnp.float32),
            pltpu.VMEM((3 * _EGC, _EC), jnp.int32),
            pltpu.VMEM((_EC, _H), jnp.float32),
            pltpu.VMEM((_EC, _H), jnp.float32),
            pltpu.VMEM((18, _H), jnp.float32),
            pltpu.SemaphoreType.DMA,
            pltpu.SemaphoreType.DMA,
        ])
    return f(h, b18, eidx)


def _dense_body(h_ref, acc_ref, e_ref, W1_ref, b1_ref, W2_ref, b2_ref,
                g_ref, bb_ref, o_ref, lin=False, W_ref=None, lb_ref=None):
    agg = acc_ref[0:_NPAD, :] + acc_ref[_NPAD:2 * _NPAD, :]
    h2 = e_ref[...] * h_ref[...] + agg
    t = jnp.maximum(
        jnp.dot(h2, W1_ref[...], preferred_element_type=jnp.float32)
        + b1_ref[...], 0.0)
    z = (jnp.dot(t, W2_ref[...], preferred_element_type=jnp.float32)
         + b2_ref[...])
    mask = lax.broadcasted_iota(jnp.int32, (_NPAD, 1), 0) < _N
    zm = jnp.where(mask, z, 0.0)
    mean = jnp.sum(zm, axis=0, keepdims=True) * (1.0 / _N)
    d = jnp.where(mask, z - mean, 0.0)
    var = jnp.sum(d * d, axis=0, keepdims=True) * (1.0 / _N)
    hb = jnp.maximum(
        g_ref[...] * (z - mean) / jnp.sqrt(var + 1e-5) + bb_ref[...], 0.0)
    if lin:
        o_ref[...] = (jnp.dot(hb, W_ref[...],
                              preferred_element_type=jnp.float32)
                      + lb_ref[...])
    else:
        o_ref[...] = hb


def _dense_call(h, accs, epsrow, W1, b1, W2, b2, gamma, beta):
    return pl.pallas_call(
        _dense_body,
        out_shape=jax.ShapeDtypeStruct((_NPAD, _H), jnp.float32),
    )(h, accs, epsrow, W1, b1, W2, b2, gamma, beta)


def _dense_final_body(h_ref, acc_ref, e_ref, W1_ref, b1_ref, W2_ref, b2_ref,
                      g_ref, bb_ref, W_ref, lb_ref, o_ref):
    _dense_body(h_ref, acc_ref, e_ref, W1_ref, b1_ref, W2_ref, b2_ref,
                g_ref, bb_ref, o_ref, lin=True, W_ref=W_ref, lb_ref=lb_ref)


def _dense_final_call(h, accs, epsrow, W1, b1, W2, b2, gamma, beta,
                      lin_W, lin_b):
    return pl.pallas_call(
        _dense_final_body,
        out_shape=jax.ShapeDtypeStruct((_NPAD, _H), jnp.float32),
    )(h, accs, epsrow, W1, b1, W2, b2, gamma, beta, lin_W, lin_b)


def kernel(x, edge_index, edge_attr, atom_emb, bond_emb, eps, W1, b1, W2, b2,
           bn_gamma, bn_beta, lin_W, lin_b):
    x = x.astype(jnp.int32)
    edge_index = edge_index.astype(jnp.int32)
    edge_attr = edge_attr.astype(jnp.int32)

    xoff = x + 100 * jnp.arange(9, dtype=jnp.int32)[None, :]
    padn = _NPAD - _N
    pad = (jnp.arange(padn * 9, dtype=jnp.int32) % 900).reshape(padn, 9)
    xfj = jnp.concatenate([xoff, pad], axis=0).reshape(
        _NPAD // _AC, _AC, 9).transpose(0, 2, 1)
    dumm = jnp.zeros((_NPAD // _AC, 16 - 9, _AC), jnp.int32)
    xf2 = jnp.concatenate([xfj, dumm], axis=1)
    atab = atom_emb.reshape(900, _H).astype(jnp.float32)

    npadE = _EP - _E
    pe = jnp.arange(npadE, dtype=jnp.int32)
    ng = _ECH // _EGC
    src3 = jnp.concatenate([edge_index[0], pe % _N]).reshape(
        _NW, ng, _EGC, _EC)
    dst3 = jnp.concatenate([edge_index[1], _N + pe % (_NPAD - _N)]).reshape(
        _NW, ng, _EGC, _EC)
    cid = (edge_attr[:, 0] * 36 + edge_attr[:, 1] * 6 + edge_attr[:, 2])
    cid3 = jnp.concatenate([cid, pe % 216]).reshape(_NW, ng, _EGC, _EC)
    eidx = jnp.concatenate([src3, cid3, dst3], axis=2)

    epsrows = (1.0 + eps.astype(jnp.float32))[:, None] * jnp.ones(
        (1, _H), jnp.float32)

    h = _atom_call(xf2, atab)
    for l in range(_L):
        b18 = bond_emb[l].reshape(18, _H).astype(jnp.float32)
        accs = _edge_call(h, b18, eidx)
        gam = bn_gamma[l].reshape(1, -1)
        bet = bn_beta[l].reshape(1, -1)
        args = (h, accs, epsrows[l:l + 1], W1[l], b1[l].reshape(1, -1),
                W2[l], b2[l].reshape(1, -1), gam, bet)
        if l < _L - 1:
            h = _dense_call(*args)
        else:
            out = _dense_final_call(*args, lin_W, lin_b.reshape(1, -1))
    return out[:_N]
